# baseline (speedup 1.0000x reference)
import functools

import jax
import jax.numpy as jnp
from jax import lax
from jax.experimental import pallas as pl
from jax.experimental.pallas import tpu as pltpu
from jax.experimental.pallas import tpu_sc as plsc

B, T, D, M = 8, 512, 256, 1024
NC, NS, L = 2, 16, 16
NW = NC * NS
ROWS = (B * M) // NW
ZROW = B * T
EPS = 1e-5


def _dp_body(x_ref, cw1_ref, cb1_ref, g1_ref, b1_ref, cw2_ref, cb2_ref,
             g2_ref, b2_ref, lw_ref, lb_ref, out_ref):
    x = x_ref[0]

    def conv_ln_relu(h, w_ref, b_row, g_row, beta_row):
        z = jnp.zeros((1, h.shape[1]), h.dtype)
        h_prev = jnp.concatenate([z, h[:-1]], axis=0)
        h_next = jnp.concatenate([h[1:], z], axis=0)
        y = (jnp.dot(h_prev, w_ref[0], preferred_element_type=jnp.float32)
             + jnp.dot(h, w_ref[1], preferred_element_type=jnp.float32)
             + jnp.dot(h_next, w_ref[2], preferred_element_type=jnp.float32)
             + b_row)
        mu = jnp.mean(y, axis=-1, keepdims=True)
        var = jnp.mean((y - mu) ** 2, axis=-1, keepdims=True)
        y = (y - mu) * lax.rsqrt(var + EPS) * g_row + beta_row
        return jnp.maximum(y, 0.0)

    h = conv_ln_relu(x, cw1_ref, cb1_ref[...], g1_ref[...], b1_ref[...])
    h = conv_ln_relu(h, cw2_ref, cb2_ref[...], g2_ref[...], b2_ref[...])
    dp = lax.dot_general(lw_ref[...], h, (((1,), (1,)), ((), ())),
                         preferred_element_type=jnp.float32)
    out_ref[...] = jnp.maximum(dp + lb_ref[0, 0], 0.0).reshape(1, 1, T)


def _duration_predictor(x, cw1, cb1, g1, b1, cw2, cb2, g2, b2, lw, lb):
    row = lambda v: v.reshape(1, -1)
    full = lambda s: pl.BlockSpec(s, lambda i: (0,) * len(s))
    return pl.pallas_call(
        _dp_body,
        grid=(B,),
        in_specs=[
            pl.BlockSpec((1, T, D), lambda i: (i, 0, 0)),
            full((3, D, D)), full((1, D)), full((1, D)), full((1, D)),
            full((3, D, D)), full((1, D)), full((1, D)), full((1, D)),
            full((1, D)), full((1, 1)),
        ],
        out_specs=pl.BlockSpec((1, 1, T), lambda i: (i, 0, 0)),
        out_shape=jax.ShapeDtypeStruct((B, 1, T), jnp.float32),
    )(x, cw1, row(cb1), row(g1), row(b1), cw2, row(cb2), row(g2), row(b2),
      lw.reshape(1, D), lb.reshape(1, 1)).reshape(B, T)


HD = D // 2
MH = M // 2


def _lr_body(x_hbm, tgt_hbm, out_hbm, tab, ends_v, cnt_v, rows_v,
             ssem, ws0, ws1):
    wsem = (ws0, ws1)
    cid = lax.axis_index("c")
    sid = lax.axis_index("s")
    wid = sid * NC + cid
    b = wid // 4
    h = (wid // 2) % 2
    mh = wid % 2

    st = pltpu.async_copy(x_hbm.at[pl.ds(b * T, T), pl.ds(h * HD, HD)],
                          tab.at[pl.ds(0, T)], ssem)

    zeros16 = jnp.zeros((L,), jnp.float32)
    for k in range(HD // L):
        tab[T, pl.ds(k * L, L)] = zeros16

    pltpu.sync_copy(tgt_hbm.at[b], ends_v)

    def cum_step(i, carry):
        chunk = ends_v[pl.ds(i * L, L)]
        ends_v[pl.ds(i * L, L)] = plsc.cumsum(chunk) + carry
        return carry + jnp.sum(chunk)

    lax.fori_loop(0, T // L, cum_step, jnp.int32(0), unroll=4)

    m0 = mh * MH

    def search_step(j, _):
        mv = m0 + j * L + lax.iota(jnp.int32, L)
        cnt = jnp.zeros((L,), jnp.int32)
        step = T
        while step >= 1:
            nc = cnt + step
            probe = jnp.minimum(nc, T) - 1
            vals = plsc.load_gather(ends_v, [probe])
            ok = (nc <= T) & (vals <= mv)
            cnt = jnp.where(ok, nc, cnt)
            step //= 2
        cnt_v[pl.ds(j * L, L)] = cnt
        return 0

    lax.fori_loop(0, MH // L, search_step, 0)

    st.wait()

    ob = b * M + m0
    row_iota = lax.iota(jnp.int32, L)
    w = [None] * 8
    for c in range(8):
        half = c % 2
        if c >= 2:
            w[c - 2].wait()
        buf = rows_v.at[half]

        def expand_j(j, _, c=c, buf=buf):
            cnt_vec = cnt_v[pl.ds((c * 4 + j) * L, L)]
            rows = j * L + row_iota

            @plsc.parallel_loop(0, HD, 8, unroll=8)
            def expand_g(g):
                for u in range(8):
                    colv = jnp.full((L,), g + u, jnp.int32)
                    v = plsc.load_gather(tab, [cnt_vec, colv])
                    plsc.store_scatter(buf, [rows, colv], v)

            return 0

        lax.fori_loop(0, 4, expand_j, 0)
        w[c] = pltpu.async_copy(
            buf, out_hbm.at[pl.ds(ob + c * 64, 64), pl.ds(h * HD, HD)],
            wsem[half])
    w[6].wait()
    w[7].wait()


@functools.cache
def _lr_kernel():
    return pl.kernel(
        _lr_body,
        out_type=jax.ShapeDtypeStruct((B * M, D), jnp.float32),
        mesh=plsc.VectorSubcoreMesh(core_axis_name="c", subcore_axis_name="s",
                                    num_cores=NC, num_subcores=NS),
        compiler_params=pltpu.CompilerParams(needs_layout_passes=False),
        scratch_types=[
            pltpu.VMEM((T + 8, HD), jnp.float32),
            pltpu.VMEM((T,), jnp.int32),
            pltpu.VMEM((MH,), jnp.int32),
            pltpu.VMEM((2, 64, HD), jnp.float32),
            pltpu.SemaphoreType.DMA,
            pltpu.SemaphoreType.DMA,
            pltpu.SemaphoreType.DMA,
        ],
    )


def kernel(x, target, mel_max_length, cw1, cb1, g1, b1, cw2, cb2, g2, b2,
           lw, lb):
    dp = _duration_predictor(x, cw1, cb1, g1, b1, cw2, cb2, g2, b2, lw, lb)
    out = _lr_kernel()(x.reshape(B * T, D),
                       target.astype(jnp.int32)).reshape(B, M, D)
    return (out, dp)


# speedup vs baseline: 1.4414x; 1.4414x over previous
import functools

import jax
import jax.numpy as jnp
from jax import lax
from jax.experimental import pallas as pl
from jax.experimental.pallas import tpu as pltpu
from jax.experimental.pallas import tpu_sc as plsc

B, T, D, M = 8, 512, 256, 1024
NC, NS, L = 2, 16, 16
NW = NC * NS
ROWS = (B * M) // NW
ZROW = B * T
EPS = 1e-5


def _dp_body(x_ref, cw1_ref, cb1_ref, g1_ref, b1_ref, cw2_ref, cb2_ref,
             g2_ref, b2_ref, lw_ref, lb_ref, out_ref):
    x = x_ref[0]

    def conv_ln_relu(h, w_ref, b_row, g_row, beta_row):
        z = jnp.zeros((1, h.shape[1]), h.dtype)
        h_prev = jnp.concatenate([z, h[:-1]], axis=0)
        h_next = jnp.concatenate([h[1:], z], axis=0)
        y = (jnp.dot(h_prev, w_ref[0], preferred_element_type=jnp.float32)
             + jnp.dot(h, w_ref[1], preferred_element_type=jnp.float32)
             + jnp.dot(h_next, w_ref[2], preferred_element_type=jnp.float32)
             + b_row)
        mu = jnp.mean(y, axis=-1, keepdims=True)
        var = jnp.mean((y - mu) ** 2, axis=-1, keepdims=True)
        y = (y - mu) * lax.rsqrt(var + EPS) * g_row + beta_row
        return jnp.maximum(y, 0.0)

    h = conv_ln_relu(x, cw1_ref, cb1_ref[...], g1_ref[...], b1_ref[...])
    h = conv_ln_relu(h, cw2_ref, cb2_ref[...], g2_ref[...], b2_ref[...])
    dp = lax.dot_general(lw_ref[...], h, (((1,), (1,)), ((), ())),
                         preferred_element_type=jnp.float32)
    out_ref[...] = jnp.maximum(dp + lb_ref[0, 0], 0.0).reshape(1, 1, T)


def _duration_predictor(x, cw1, cb1, g1, b1, cw2, cb2, g2, b2, lw, lb):
    row = lambda v: v.reshape(1, -1)
    full = lambda s: pl.BlockSpec(s, lambda i: (0,) * len(s))
    return pl.pallas_call(
        _dp_body,
        grid=(B,),
        in_specs=[
            pl.BlockSpec((1, T, D), lambda i: (i, 0, 0)),
            full((3, D, D)), full((1, D)), full((1, D)), full((1, D)),
            full((3, D, D)), full((1, D)), full((1, D)), full((1, D)),
            full((1, D)), full((1, 1)),
        ],
        out_specs=pl.BlockSpec((1, 1, T), lambda i: (i, 0, 0)),
        out_shape=jax.ShapeDtypeStruct((B, 1, T), jnp.float32),
    )(x, cw1, row(cb1), row(g1), row(b1), cw2, row(cb2), row(g2), row(b2),
      lw.reshape(1, D), lb.reshape(1, 1)).reshape(B, T)


HD = D // 2
MH = M // 2


def _lr_body(x_hbm, tgt_hbm, out_hbm, tab, ends_v, cnt_v, rows_v,
             ssem, ws0, ws1):
    wsem = (ws0, ws1)
    cid = lax.axis_index("c")
    sid = lax.axis_index("s")
    wid = sid * NC + cid
    b = wid // 4
    h = (wid // 2) % 2
    mh = wid % 2

    st = pltpu.async_copy(x_hbm.at[pl.ds(b * T, T), pl.ds(h * HD, HD)],
                          tab.at[pl.ds(0, T)], ssem)

    zeros16 = jnp.zeros((L,), jnp.float32)
    for k in range(HD // L):
        tab[T, pl.ds(k * L, L)] = zeros16

    pltpu.sync_copy(tgt_hbm.at[b], ends_v)

    def cum_step(i, carry):
        chunk = ends_v[pl.ds(i * L, L)]
        ends_v[pl.ds(i * L, L)] = plsc.cumsum(chunk) + carry
        return carry + jnp.sum(chunk)

    lax.fori_loop(0, T // L, cum_step, jnp.int32(0), unroll=4)

    m0 = mh * MH

    def search_step(j, _):
        mv = m0 + j * L + lax.iota(jnp.int32, L)
        cnt = jnp.zeros((L,), jnp.int32)
        step = T
        while step >= 1:
            nc = cnt + step
            probe = jnp.minimum(nc, T) - 1
            vals = plsc.load_gather(ends_v, [probe])
            ok = (nc <= T) & (vals <= mv)
            cnt = jnp.where(ok, nc, cnt)
            step //= 2
        cnt_v[pl.ds(j * L, L)] = cnt
        return 0

    lax.fori_loop(0, MH // L, search_step, 0)

    st.wait()

    ob = b * M + m0
    row_iota = lax.iota(jnp.int32, L)
    w = [None] * 8
    for c in range(8):
        half = c % 2
        if c >= 2:
            w[c - 2].wait()
        buf = rows_v.at[half]

        @plsc.parallel_loop(0, 4, 1)
        def expand_j(j, c=c, buf=buf):
            cnt_vec = cnt_v[pl.ds((c * 4 + j) * L, L)]
            for l in range(L):
                r = cnt_vec[l]
                row = j * L + l
                for k in range(HD // L):
                    buf[row, pl.ds(k * L, L)] = tab[r, pl.ds(k * L, L)]
        w[c] = pltpu.async_copy(
            buf, out_hbm.at[pl.ds(ob + c * 64, 64), pl.ds(h * HD, HD)],
            wsem[half])
    w[6].wait()
    w[7].wait()


@functools.cache
def _lr_kernel():
    return pl.kernel(
        _lr_body,
        out_type=jax.ShapeDtypeStruct((B * M, D), jnp.float32),
        mesh=plsc.VectorSubcoreMesh(core_axis_name="c", subcore_axis_name="s",
                                    num_cores=NC, num_subcores=NS),
        compiler_params=pltpu.CompilerParams(needs_layout_passes=False),
        scratch_types=[
            pltpu.VMEM((T + 8, HD), jnp.float32),
            pltpu.VMEM((T,), jnp.int32),
            pltpu.VMEM((MH,), jnp.int32),
            pltpu.VMEM((2, 64, HD), jnp.float32),
            pltpu.SemaphoreType.DMA,
            pltpu.SemaphoreType.DMA,
            pltpu.SemaphoreType.DMA,
        ],
    )


def kernel(x, target, mel_max_length, cw1, cb1, g1, b1, cw2, cb2, g2, b2,
           lw, lb):
    dp = _duration_predictor(x, cw1, cb1, g1, b1, cw2, cb2, g2, b2, lw, lb)
    out = _lr_kernel()(x.reshape(B * T, D),
                       target.astype(jnp.int32)).reshape(B, M, D)
    return (out, dp)


# speedup vs baseline: 1.7303x; 1.2004x over previous
import functools

import jax
import jax.numpy as jnp
from jax import lax
from jax.experimental import pallas as pl
from jax.experimental.pallas import tpu as pltpu
from jax.experimental.pallas import tpu_sc as plsc

B, T, D, M = 8, 512, 256, 1024
NC, NS, L = 2, 16, 16
NW = NC * NS
ROWS = (B * M) // NW
ZROW = B * T
EPS = 1e-5


def _dp_body(x_ref, cw1_ref, cb1_ref, g1_ref, b1_ref, cw2_ref, cb2_ref,
             g2_ref, b2_ref, lw_ref, lb_ref, out_ref):
    x = x_ref[0]

    def conv_ln_relu(h, w_ref, b_row, g_row, beta_row):
        z = jnp.zeros((1, h.shape[1]), h.dtype)
        h_prev = jnp.concatenate([z, h[:-1]], axis=0)
        h_next = jnp.concatenate([h[1:], z], axis=0)
        y = (jnp.dot(h_prev, w_ref[0], preferred_element_type=jnp.float32)
             + jnp.dot(h, w_ref[1], preferred_element_type=jnp.float32)
             + jnp.dot(h_next, w_ref[2], preferred_element_type=jnp.float32)
             + b_row)
        mu = jnp.mean(y, axis=-1, keepdims=True)
        var = jnp.mean((y - mu) ** 2, axis=-1, keepdims=True)
        y = (y - mu) * lax.rsqrt(var + EPS) * g_row + beta_row
        return jnp.maximum(y, 0.0)

    h = conv_ln_relu(x, cw1_ref, cb1_ref[...], g1_ref[...], b1_ref[...])
    h = conv_ln_relu(h, cw2_ref, cb2_ref[...], g2_ref[...], b2_ref[...])
    dp = lax.dot_general(lw_ref[...], h, (((1,), (1,)), ((), ())),
                         preferred_element_type=jnp.float32)
    out_ref[...] = jnp.maximum(dp + lb_ref[0, 0], 0.0).reshape(1, 1, T)


def _duration_predictor(x, cw1, cb1, g1, b1, cw2, cb2, g2, b2, lw, lb):
    row = lambda v: v.reshape(1, -1)
    full = lambda s: pl.BlockSpec(s, lambda i: (0,) * len(s))
    return pl.pallas_call(
        _dp_body,
        grid=(B,),
        in_specs=[
            pl.BlockSpec((1, T, D), lambda i: (i, 0, 0)),
            full((3, D, D)), full((1, D)), full((1, D)), full((1, D)),
            full((3, D, D)), full((1, D)), full((1, D)), full((1, D)),
            full((1, D)), full((1, 1)),
        ],
        out_specs=pl.BlockSpec((1, 1, T), lambda i: (i, 0, 0)),
        out_shape=jax.ShapeDtypeStruct((B, 1, T), jnp.float32),
    )(x, cw1, row(cb1), row(g1), row(b1), cw2, row(cb2), row(g2), row(b2),
      lw.reshape(1, D), lb.reshape(1, 1)).reshape(B, T)


HD = D // 2
MH = M // 2


def _lr_body(x_hbm, tgt_hbm, out_hbm, tab, ends_v, cnt_v, rows_v,
             ssem, ws0, ws1):
    wsem = (ws0, ws1)
    cid = lax.axis_index("c")
    sid = lax.axis_index("s")
    wid = sid * NC + cid
    b = wid // 4
    h = (wid // 2) % 2
    mh = wid % 2

    st = pltpu.async_copy(x_hbm.at[pl.ds(b * T, T), pl.ds(h * HD, HD)],
                          tab.at[pl.ds(0, T)], ssem)

    zeros16 = jnp.zeros((L,), jnp.float32)
    for k in range(HD // L):
        tab[T, pl.ds(k * L, L)] = zeros16

    pltpu.sync_copy(tgt_hbm.at[b], ends_v)

    def cum_step(i, carry):
        chunk = ends_v[pl.ds(i * L, L)]
        ends_v[pl.ds(i * L, L)] = plsc.cumsum(chunk) + carry
        return carry + jnp.sum(chunk)

    lax.fori_loop(0, T // L, cum_step, jnp.int32(0), unroll=4)

    m0 = mh * MH

    def search_step(j, _):
        mv = m0 + j * L + lax.iota(jnp.int32, L)
        cnt = jnp.zeros((L,), jnp.int32)
        step = T
        while step >= 1:
            nc = cnt + step
            probe = jnp.minimum(nc, T) - 1
            vals = plsc.load_gather(ends_v, [probe])
            ok = (nc <= T) & (vals <= mv)
            cnt = jnp.where(ok, nc, cnt)
            step //= 2
        cnt_v[pl.ds(j * L, L)] = cnt
        return 0

    lax.fori_loop(0, MH // L, search_step, 0)

    st.wait()

    ob = b * M + m0

    def chunk_pair(c2, _):
        for half in range(2):
            c = c2 * 2 + half
            buf = rows_v.at[half]

            @pl.when(c2 > 0)
            def _drain(half=half, buf=buf):
                pltpu.make_async_copy(
                    x_hbm.at[pl.ds(0, 64), pl.ds(0, HD)], buf,
                    wsem[half]).wait()

            @plsc.parallel_loop(0, 4, 1)
            def expand_j(j, c=c, buf=buf):
                cnt_vec = cnt_v[pl.ds((c * 4 + j) * L, L)]
                for l in range(L):
                    r = cnt_vec[l]
                    row = j * L + l
                    for k in range(HD // L):
                        buf[row, pl.ds(k * L, L)] = tab[r, pl.ds(k * L, L)]

            pltpu.async_copy(
                buf, out_hbm.at[pl.ds(ob + c * 64, 64), pl.ds(h * HD, HD)],
                wsem[half])
        return 0

    lax.fori_loop(0, 4, chunk_pair, 0)
    for half in range(2):
        pltpu.make_async_copy(x_hbm.at[pl.ds(0, 64), pl.ds(0, HD)],
                              rows_v.at[half], wsem[half]).wait()


@functools.cache
def _lr_kernel():
    return pl.kernel(
        _lr_body,
        out_type=jax.ShapeDtypeStruct((B * M, D), jnp.float32),
        mesh=plsc.VectorSubcoreMesh(core_axis_name="c", subcore_axis_name="s",
                                    num_cores=NC, num_subcores=NS),
        compiler_params=pltpu.CompilerParams(needs_layout_passes=False),
        scratch_types=[
            pltpu.VMEM((T + 8, HD), jnp.float32),
            pltpu.VMEM((T,), jnp.int32),
            pltpu.VMEM((MH,), jnp.int32),
            pltpu.VMEM((2, 64, HD), jnp.float32),
            pltpu.SemaphoreType.DMA,
            pltpu.SemaphoreType.DMA,
            pltpu.SemaphoreType.DMA,
        ],
    )


def kernel(x, target, mel_max_length, cw1, cb1, g1, b1, cw2, cb2, g2, b2,
           lw, lb):
    dp = _duration_predictor(x, cw1, cb1, g1, b1, cw2, cb2, g2, b2, lw, lb)
    out = _lr_kernel()(x.reshape(B * T, D),
                       target.astype(jnp.int32)).reshape(B, M, D)
    return (out, dp)
